# baseline (speedup 1.0000x reference)
import functools

import jax
import jax.numpy as jnp
from jax import lax
from jax.experimental import pallas as pl
from jax.experimental.pallas import tpu as pltpu
from jax.experimental.pallas import tpu_sc as plsc

N = 10000
E = 320000
D = 128

NC = 2
NS = 16
NW = NC * NS
CHUNK = 128
NCHUNK = 80
E_PAD = NW * NCHUNK * CHUNK
NBUF = 2
NIDX = 4
NPAD = 10112
ROWS_PER_SUBCORE = NPAD // NS


def _segment_sum_sc(h, src, dst):
    mesh = plsc.VectorSubcoreMesh(core_axis_name="c", subcore_axis_name="s")

    @functools.partial(
        pl.kernel,
        out_type=jax.ShapeDtypeStruct((NC, N, D), jnp.float32),
        mesh=mesh,
        scratch_types=[
            pltpu.VMEM((CHUNK,), jnp.int32),
            pltpu.VMEM((CHUNK,), jnp.int32),
            pltpu.VMEM((CHUNK,), jnp.int32),
            pltpu.VMEM((CHUNK,), jnp.int32),
            pltpu.VMEM((CHUNK, D), jnp.float32),
            pltpu.VMEM((CHUNK, D), jnp.float32),
            pltpu.VMEM_SHARED((NPAD, D), jnp.float32),
            pltpu.SemaphoreType.DMA,
            pltpu.SemaphoreType.DMA,
        ],
    )
    def seg_kernel(h_hbm, src_hbm, dst_hbm, out_hbm,
                   srcA, dstA, srcB, dstB, rows0, rows1, table,
                   semA, semB):
        cid = lax.axis_index("c")
        sid = lax.axis_index("s")
        wid = sid * NC + cid

        @pl.loop(0, CHUNK)
        def _(r):
            @pl.loop(0, D, step=16)
            def _(c0):
                rows0.at[r, pl.ds(c0, 16)][...] = jnp.zeros(
                    (16,), jnp.float32)

        row0 = sid * ROWS_PER_SUBCORE
        for k in range(ROWS_PER_SUBCORE // CHUNK):
            pltpu.sync_copy(rows0,
                            table.at[pl.ds(row0 + k * CHUNK, CHUNK)])
        _rem = ROWS_PER_SUBCORE % CHUNK
        pltpu.sync_copy(
            rows0.at[pl.ds(0, _rem)],
            table.at[pl.ds(row0 + ROWS_PER_SUBCORE - _rem, _rem)])
        plsc.subcore_barrier()

        @pl.loop(wid * 2 * CHUNK, E, step=NW * 2 * CHUNK)
        def _(e0):
            e1 = e0 + CHUNK
            pltpu.sync_copy(src_hbm.at[pl.ds(e0, CHUNK)], srcA)
            pltpu.sync_copy(dst_hbm.at[pl.ds(e0, CHUNK)], dstA)
            hA = pltpu.async_copy(h_hbm.at[srcA], rows0, semA)
            pltpu.sync_copy(src_hbm.at[pl.ds(e1, CHUNK)], srcB)
            pltpu.sync_copy(dst_hbm.at[pl.ds(e1, CHUNK)], dstB)
            hB = pltpu.async_copy(h_hbm.at[srcB], rows1, semB)
            hA.wait()
            sA = pltpu.async_copy(rows0, table.at[dstA], semA, add=True)
            hB.wait()
            pltpu.sync_copy(rows1, table.at[dstB], add=True)
            sA.wait()

        plsc.subcore_barrier()

        @pl.when(row0 + ROWS_PER_SUBCORE <= N)
        def _():
            pltpu.sync_copy(table.at[pl.ds(row0, ROWS_PER_SUBCORE)],
                            out_hbm.at[cid, pl.ds(row0, ROWS_PER_SUBCORE)])

        @pl.when(row0 + ROWS_PER_SUBCORE > N)
        def _():
            pltpu.sync_copy(table.at[pl.ds(row0, N % ROWS_PER_SUBCORE)],
                            out_hbm.at[cid, pl.ds(row0, N % ROWS_PER_SUBCORE)])

    return seg_kernel(h, src, dst)


_BLK = 1000


def _mlp_body(eps_ref, h_ref, agg_ref, w1_ref, b1_ref, w2_ref, b2_ref, o_ref):
    z = (1.0 + eps_ref[0]) * h_ref[...] + agg_ref[0] + agg_ref[1]
    t = jnp.maximum(
        jnp.dot(z, w1_ref[...], preferred_element_type=jnp.float32)
        + b1_ref[...], 0.0)
    o = jnp.maximum(
        jnp.dot(t, w2_ref[...], preferred_element_type=jnp.float32)
        + b2_ref[...], 0.0)
    o_ref[...] = o


def _gin_mlp_tc(h, agg, W1, b1, W2, b2, eps):
    grid = (N // _BLK,)
    return pl.pallas_call(
        _mlp_body,
        grid=grid,
        in_specs=[
            pl.BlockSpec(memory_space=pltpu.SMEM),
            pl.BlockSpec((_BLK, D), lambda i: (i, 0)),
            pl.BlockSpec((NC, _BLK, D), lambda i: (0, i, 0)),
            pl.BlockSpec((D, 2 * D), lambda i: (0, 0)),
            pl.BlockSpec((1, 2 * D), lambda i: (0, 0)),
            pl.BlockSpec((2 * D, D), lambda i: (0, 0)),
            pl.BlockSpec((1, D), lambda i: (0, 0)),
        ],
        out_specs=pl.BlockSpec((_BLK, D), lambda i: (i, 0)),
        out_shape=jax.ShapeDtypeStruct((N, D), jnp.float32),
    )(eps.reshape(1), h, agg, W1, b1.reshape(1, -1), W2, b2.reshape(1, -1))


def _final_body(h0_ref, h1_ref, h2_ref, h3_ref, w_ref, b_ref, o_ref):
    w = w_ref[...]
    o = jnp.dot(h0_ref[...], w[0 * D:1 * D], preferred_element_type=jnp.float32)
    o += jnp.dot(h1_ref[...], w[1 * D:2 * D], preferred_element_type=jnp.float32)
    o += jnp.dot(h2_ref[...], w[2 * D:3 * D], preferred_element_type=jnp.float32)
    o += jnp.dot(h3_ref[...], w[3 * D:4 * D], preferred_element_type=jnp.float32)
    o_ref[...] = o + b_ref[...]


def _final_linear_tc(h0, h1, h2, h3, lin_W, lin_b):
    grid = (N // _BLK,)
    row_spec = pl.BlockSpec((_BLK, D), lambda i: (i, 0))
    return pl.pallas_call(
        _final_body,
        grid=grid,
        in_specs=[
            row_spec, row_spec, row_spec, row_spec,
            pl.BlockSpec((4 * D, D), lambda i: (0, 0)),
            pl.BlockSpec((1, D), lambda i: (0, 0)),
        ],
        out_specs=row_spec,
        out_shape=jax.ShapeDtypeStruct((N, D), jnp.float32),
    )(h0, h1, h2, h3, lin_W, lin_b.reshape(1, -1))


def kernel(x, edge_index, W1_0, b1_0, W2_0, b2_0, eps_0,
           W1_1, b1_1, W2_1, b2_1, eps_1,
           W1_2, b1_2, W2_2, b2_2, eps_2, lin_W, lin_b):
    params = [
        (W1_0, b1_0, W2_0, b2_0, eps_0),
        (W1_1, b1_1, W2_1, b2_1, eps_1),
        (W1_2, b1_2, W2_2, b2_2, eps_2),
    ]
    src = edge_index[0]
    dst = edge_index[1]
    h = x
    h_list = [x]
    for (W1, b1, W2, b2, eps) in params:
        agg = _segment_sum_sc(h, src, dst)
        h = _gin_mlp_tc(h, agg, W1, b1, W2, b2, eps)
        h_list.append(h)
    return _final_linear_tc(h_list[0], h_list[1], h_list[2], h_list[3],
                            lin_W, lin_b)


# speedup vs baseline: 1.0039x; 1.0039x over previous
import functools

import jax
import jax.numpy as jnp
from jax import lax
from jax.experimental import pallas as pl
from jax.experimental.pallas import tpu as pltpu
from jax.experimental.pallas import tpu_sc as plsc

N = 10000
E = 320000
D = 128

NC = 2
NS = 16
NW = NC * NS
CHUNK = 128
NPAD = 10112
ROWS_PER_SUBCORE = NPAD // NS


def _segment_sum_sc(h, src, dst):
    mesh = plsc.VectorSubcoreMesh(core_axis_name="c", subcore_axis_name="s")

    @functools.partial(
        pl.kernel,
        out_type=jax.ShapeDtypeStruct((NC, N, D), jnp.float32),
        mesh=mesh,
        scratch_types=[
            pltpu.VMEM((CHUNK,), jnp.int32),
            pltpu.VMEM((CHUNK,), jnp.int32),
            pltpu.VMEM((CHUNK,), jnp.int32),
            pltpu.VMEM((CHUNK,), jnp.int32),
            pltpu.VMEM((CHUNK, D), jnp.float32),
            pltpu.VMEM((CHUNK, D), jnp.float32),
            pltpu.VMEM_SHARED((NPAD, D), jnp.float32),
            pltpu.SemaphoreType.DMA,
            pltpu.SemaphoreType.DMA,
        ],
    )
    def seg_kernel(h_hbm, src_hbm, dst_hbm, out_hbm,
                   srcA, dstA, srcB, dstB, rows0, rows1, table,
                   semA, semB):
        cid = lax.axis_index("c")
        sid = lax.axis_index("s")
        wid = sid * NC + cid

        @pl.loop(0, CHUNK)
        def _(r):
            @pl.loop(0, D, step=16)
            def _(c0):
                rows0.at[r, pl.ds(c0, 16)][...] = jnp.zeros(
                    (16,), jnp.float32)

        row0 = sid * ROWS_PER_SUBCORE
        for k in range(ROWS_PER_SUBCORE // CHUNK):
            pltpu.sync_copy(rows0,
                            table.at[pl.ds(row0 + k * CHUNK, CHUNK)])
        _rem = ROWS_PER_SUBCORE % CHUNK
        pltpu.sync_copy(
            rows0.at[pl.ds(0, _rem)],
            table.at[pl.ds(row0 + ROWS_PER_SUBCORE - _rem, _rem)])
        plsc.subcore_barrier()

        @pl.loop(wid * 2 * CHUNK, E, step=NW * 2 * CHUNK)
        def _(e0):
            e1 = e0 + CHUNK
            pltpu.sync_copy(src_hbm.at[pl.ds(e0, CHUNK)], srcA)
            pltpu.sync_copy(dst_hbm.at[pl.ds(e0, CHUNK)], dstA)
            hA = pltpu.async_copy(h_hbm.at[srcA], rows0, semA)
            pltpu.sync_copy(src_hbm.at[pl.ds(e1, CHUNK)], srcB)
            pltpu.sync_copy(dst_hbm.at[pl.ds(e1, CHUNK)], dstB)
            hB = pltpu.async_copy(h_hbm.at[srcB], rows1, semB)
            hA.wait()
            pltpu.sync_copy(rows0, table.at[dstA], add=True)
            hB.wait()
            pltpu.sync_copy(rows1, table.at[dstB], add=True)

        plsc.subcore_barrier()

        @pl.when(row0 + ROWS_PER_SUBCORE <= N)
        def _():
            pltpu.sync_copy(table.at[pl.ds(row0, ROWS_PER_SUBCORE)],
                            out_hbm.at[cid, pl.ds(row0, ROWS_PER_SUBCORE)])

        @pl.when(row0 + ROWS_PER_SUBCORE > N)
        def _():
            pltpu.sync_copy(table.at[pl.ds(row0, N % ROWS_PER_SUBCORE)],
                            out_hbm.at[cid, pl.ds(row0, N % ROWS_PER_SUBCORE)])

    return seg_kernel(h, src, dst)


_BLK = 1000


def _mlp_body(eps_ref, h_ref, agg_ref, w1_ref, b1_ref, w2_ref, b2_ref, o_ref):
    z = (1.0 + eps_ref[0]) * h_ref[...] + agg_ref[0] + agg_ref[1]
    t = jnp.maximum(
        jnp.dot(z, w1_ref[...], preferred_element_type=jnp.float32)
        + b1_ref[...], 0.0)
    o = jnp.maximum(
        jnp.dot(t, w2_ref[...], preferred_element_type=jnp.float32)
        + b2_ref[...], 0.0)
    o_ref[...] = o


def _gin_mlp_tc(h, agg, W1, b1, W2, b2, eps):
    grid = (N // _BLK,)
    return pl.pallas_call(
        _mlp_body,
        grid=grid,
        in_specs=[
            pl.BlockSpec(memory_space=pltpu.SMEM),
            pl.BlockSpec((_BLK, D), lambda i: (i, 0)),
            pl.BlockSpec((NC, _BLK, D), lambda i: (0, i, 0)),
            pl.BlockSpec((D, 2 * D), lambda i: (0, 0)),
            pl.BlockSpec((1, 2 * D), lambda i: (0, 0)),
            pl.BlockSpec((2 * D, D), lambda i: (0, 0)),
            pl.BlockSpec((1, D), lambda i: (0, 0)),
        ],
        out_specs=pl.BlockSpec((_BLK, D), lambda i: (i, 0)),
        out_shape=jax.ShapeDtypeStruct((N, D), jnp.float32),
    )(eps.reshape(1), h, agg, W1, b1.reshape(1, -1), W2, b2.reshape(1, -1))


def _final_body(h0_ref, h1_ref, h2_ref, h3_ref, w_ref, b_ref, o_ref):
    w = w_ref[...]
    o = jnp.dot(h0_ref[...], w[0 * D:1 * D], preferred_element_type=jnp.float32)
    o += jnp.dot(h1_ref[...], w[1 * D:2 * D], preferred_element_type=jnp.float32)
    o += jnp.dot(h2_ref[...], w[2 * D:3 * D], preferred_element_type=jnp.float32)
    o += jnp.dot(h3_ref[...], w[3 * D:4 * D], preferred_element_type=jnp.float32)
    o_ref[...] = o + b_ref[...]


def _final_linear_tc(h0, h1, h2, h3, lin_W, lin_b):
    grid = (N // _BLK,)
    row_spec = pl.BlockSpec((_BLK, D), lambda i: (i, 0))
    return pl.pallas_call(
        _final_body,
        grid=grid,
        in_specs=[
            row_spec, row_spec, row_spec, row_spec,
            pl.BlockSpec((4 * D, D), lambda i: (0, 0)),
            pl.BlockSpec((1, D), lambda i: (0, 0)),
        ],
        out_specs=row_spec,
        out_shape=jax.ShapeDtypeStruct((N, D), jnp.float32),
    )(h0, h1, h2, h3, lin_W, lin_b.reshape(1, -1))


def kernel(x, edge_index, W1_0, b1_0, W2_0, b2_0, eps_0,
           W1_1, b1_1, W2_1, b2_1, eps_1,
           W1_2, b1_2, W2_2, b2_2, eps_2, lin_W, lin_b):
    params = [
        (W1_0, b1_0, W2_0, b2_0, eps_0),
        (W1_1, b1_1, W2_1, b2_1, eps_1),
        (W1_2, b1_2, W2_2, b2_2, eps_2),
    ]
    src = edge_index[0]
    dst = edge_index[1]
    h = x
    h_list = [x]
    for (W1, b1, W2, b2, eps) in params:
        agg = _segment_sum_sc(h, src, dst)
        h = _gin_mlp_tc(h, agg, W1, b1, W2, b2, eps)
        h_list.append(h)
    return _final_linear_tc(h_list[0], h_list[1], h_list[2], h_list[3],
                            lin_W, lin_b)
